# Initial kernel scaffold; baseline (speedup 1.0000x reference)
import numpy as np
import jax
import jax.numpy as jnp
from jax import lax
from jax.experimental import pallas as pl
from jax.experimental.pallas import tpu as pltpu
from jax.experimental.pallas import tpu_sc as plsc

_N_LEVELS = 16
_LOG2_T = 19
_T = 1 << _LOG2_T
_BASE_RES = 16
_SCALE = 1.26
_N = 1048576
_PRIME1 = np.int32(np.uint32(2654435761).view(np.int32))
_PRIME2 = np.int32(805459861)
_RES = [int(np.ceil(_BASE_RES * (_SCALE ** l))) for l in range(_N_LEVELS)]
_DENSE = [(r + 1) ** 3 <= _T for r in _RES]

_NC, _NS, _L = 2, 16, 16
_NW = _NC * _NS
_PTS_PER_TILE = _N // _NW
_P = 2048
_CHUNKS = _PTS_PER_TILE // _P
_STRIPS = _P // _L
_NIDX = 8 * _P


def _encode_body(xT, tab, encT, x0, x1, x2, idxb, wcb, rows, f0a, f1a, sem):
    wid = lax.axis_index("s") * _NC + lax.axis_index("c")
    tile_base = wid * _PTS_PER_TILE

    @pl.loop(0, _CHUNKS)
    def _chunk(ci):
        base = tile_base + ci * _P
        pltpu.sync_copy(xT.at[0, pl.ds(base, _P)], x0)
        pltpu.sync_copy(xT.at[1, pl.ds(base, _P)], x1)
        pltpu.sync_copy(xT.at[2, pl.ds(base, _P)], x2)

        for l in range(_N_LEVELS):
            res = _RES[l]
            dense = _DENSE[l]
            fres = jnp.float32(res)

            @pl.loop(0, _STRIPS)
            def _pa(s):
                o = s * _L
                xs = (x0[pl.ds(o, _L)], x1[pl.ds(o, _L)], x2[pl.ds(o, _L)])
                pis = []
                wlo = []
                whi = []
                for d in range(3):
                    pos = xs[d] * fres
                    pi = pos.astype(jnp.int32)
                    fr = pos - pi.astype(jnp.float32)
                    w = fr * fr * (3.0 - 2.0 * fr)
                    pis.append(pi)
                    whi.append(w)
                    wlo.append(1.0 - w)
                for corner in range(8):
                    bits = [(corner >> d) & 1 for d in range(3)]
                    c0 = pis[0] + bits[0] if bits[0] else pis[0]
                    c1 = pis[1] + bits[1] if bits[1] else pis[1]
                    c2 = pis[2] + bits[2] if bits[2] else pis[2]
                    if dense:
                        idx = c0 + c1 * (res + 1) + c2 * ((res + 1) * (res + 1))
                    else:
                        idx = (c0 ^ (c1 * _PRIME1) ^ (c2 * _PRIME2)) & (_T - 1)
                    wc = ((whi[0] if bits[0] else wlo[0])
                          * (whi[1] if bits[1] else wlo[1])
                          * (whi[2] if bits[2] else wlo[2]))
                    idxb[pl.ds(corner * _P + o, _L)] = idx + (l * _T)
                    wcb[pl.ds(corner * _P + o, _L)] = wc

            pltpu.async_copy(tab.at[idxb], rows, sem).wait()

            @pl.loop(0, _STRIPS)
            def _pc(s):
                o = s * _L
                iota = lax.iota(jnp.int32, _L)
                col0 = jnp.zeros((_L,), jnp.int32)
                col1 = col0 + 1
                acc0 = jnp.zeros((_L,), jnp.float32)
                acc1 = jnp.zeros((_L,), jnp.float32)
                for corner in range(8):
                    rb = iota + (corner * _P + o)
                    wc = wcb[pl.ds(corner * _P + o, _L)]
                    f0 = plsc.load_gather(rows, [rb, col0])
                    f1 = plsc.load_gather(rows, [rb, col1])
                    acc0 = acc0 + f0 * wc
                    acc1 = acc1 + f1 * wc
                f0a[pl.ds(o, _L)] = acc0
                f1a[pl.ds(o, _L)] = acc1

            pltpu.sync_copy(f0a, encT.at[2 * l, pl.ds(base, _P)])
            pltpu.sync_copy(f1a, encT.at[2 * l + 1, pl.ds(base, _P)])


def _encode(xT, tab):
    mesh = plsc.VectorSubcoreMesh(core_axis_name="c", subcore_axis_name="s")
    return pl.kernel(
        _encode_body,
        out_type=jax.ShapeDtypeStruct((2 * _N_LEVELS, _N), jnp.float32),
        mesh=mesh,
        scratch_types=[
            pltpu.VMEM((_P,), jnp.float32),
            pltpu.VMEM((_P,), jnp.float32),
            pltpu.VMEM((_P,), jnp.float32),
            pltpu.VMEM((_NIDX,), jnp.int32),
            pltpu.VMEM((_NIDX,), jnp.float32),
            pltpu.VMEM((_NIDX, 2), jnp.float32),
            pltpu.VMEM((_P,), jnp.float32),
            pltpu.VMEM((_P,), jnp.float32),
            pltpu.SemaphoreType.DMA,
        ],
    )(xT, tab)


_MLP_B = 2048


def _mlp_body(enc_ref, w1t_ref, w2t_ref, out_ref):
    e = enc_ref[...]
    h = jnp.maximum(jnp.dot(w1t_ref[...], e, preferred_element_type=jnp.float32), 0.0)
    o = jnp.dot(w2t_ref[...], h, preferred_element_type=jnp.float32)
    out_ref[...] = jnp.clip(o, 0.0, 1.0)


def _mlp(encT, W1T, W2Tp):
    return pl.pallas_call(
        _mlp_body,
        grid=(_N // _MLP_B,),
        in_specs=[
            pl.BlockSpec((2 * _N_LEVELS, _MLP_B), lambda i: (0, i)),
            pl.BlockSpec((64, 32), lambda i: (0, 0)),
            pl.BlockSpec((8, 64), lambda i: (0, 0)),
        ],
        out_specs=pl.BlockSpec((8, _MLP_B), lambda i: (0, i)),
        out_shape=jax.ShapeDtypeStruct((8, _N), jnp.float32),
    )(encT, W1T, W2Tp)


def kernel(x, table, W1, W2):
    xT = x.T
    tab = table.reshape(_N_LEVELS * _T, 2)
    encT = _encode(xT, tab)
    W1T = W1.T
    W2Tp = jnp.zeros((8, 64), jnp.float32).at[:3].set(W2.T)
    outT = _mlp(encT, W1T, W2Tp)
    return outT[:3].T


# speedup vs baseline: 41.5667x; 41.5667x over previous
import numpy as np
import jax
import jax.numpy as jnp
from jax import lax
from jax.experimental import pallas as pl
from jax.experimental.pallas import tpu as pltpu
from jax.experimental.pallas import tpu_sc as plsc

_N_LEVELS = 16
_LOG2_T = 19
_T = 1 << _LOG2_T
_BASE_RES = 16
_SCALE = 1.26
_N = 1048576
_PRIME1 = np.int32(np.uint32(2654435761).view(np.int32))
_PRIME2 = np.int32(805459861)
_RES = [int(np.ceil(_BASE_RES * (_SCALE ** l))) for l in range(_N_LEVELS)]
_DENSE = [(r + 1) ** 3 <= _T for r in _RES]

_NC, _NS, _L = 2, 16, 16
_NW = _NC * _NS
_PTS_PER_TILE = _N // _NW
_P = 2048
_CHUNKS = _PTS_PER_TILE // _P
_STRIPS = _P // _L
_NIDX = 8 * _P


def _encode_body(xT, tab, encT, x0, x1, x2, idxb0, idxb1, wcb, rf0, rf1,
                 f0a, f1a, sem):
    wid = lax.axis_index("s") * _NC + lax.axis_index("c")
    tile_base = wid * _PTS_PER_TILE

    @pl.loop(0, _CHUNKS)
    def _chunk(ci):
        base = tile_base + ci * _P
        pltpu.sync_copy(xT.at[0, pl.ds(base, _P)], x0)
        pltpu.sync_copy(xT.at[1, pl.ds(base, _P)], x1)
        pltpu.sync_copy(xT.at[2, pl.ds(base, _P)], x2)

        for l in range(_N_LEVELS):
            res = _RES[l]
            dense = _DENSE[l]
            fres = jnp.float32(res)

            @pl.loop(0, _STRIPS)
            def _pa(s):
                o = s * _L
                xs = (x0[pl.ds(o, _L)], x1[pl.ds(o, _L)], x2[pl.ds(o, _L)])
                pis = []
                wlo = []
                whi = []
                for d in range(3):
                    pos = xs[d] * fres
                    pi = pos.astype(jnp.int32)
                    fr = pos - pi.astype(jnp.float32)
                    w = fr * fr * (3.0 - 2.0 * fr)
                    pis.append(pi)
                    whi.append(w)
                    wlo.append(1.0 - w)
                for corner in range(8):
                    bits = [(corner >> d) & 1 for d in range(3)]
                    c0 = pis[0] + bits[0] if bits[0] else pis[0]
                    c1 = pis[1] + bits[1] if bits[1] else pis[1]
                    c2 = pis[2] + bits[2] if bits[2] else pis[2]
                    if dense:
                        idx = c0 + c1 * (res + 1) + c2 * ((res + 1) * (res + 1))
                    else:
                        idx = (c0 ^ (c1 * _PRIME1) ^ (c2 * _PRIME2)) & (_T - 1)
                    wc = ((whi[0] if bits[0] else wlo[0])
                          * (whi[1] if bits[1] else wlo[1])
                          * (whi[2] if bits[2] else wlo[2]))
                    g2 = (idx + (l * _T)) * 2
                    idxb0[pl.ds(corner * _P + o, _L)] = g2
                    idxb1[pl.ds(corner * _P + o, _L)] = g2 + 1
                    wcb[pl.ds(corner * _P + o, _L)] = wc

            cp0 = pltpu.async_copy(tab.at[idxb0], rf0, sem)
            cp1 = pltpu.async_copy(tab.at[idxb1], rf1, sem)
            cp0.wait()
            cp1.wait()

            @pl.loop(0, _STRIPS)
            def _pc(s):
                o = s * _L
                acc0 = jnp.zeros((_L,), jnp.float32)
                acc1 = jnp.zeros((_L,), jnp.float32)
                for corner in range(8):
                    wc = wcb[pl.ds(corner * _P + o, _L)]
                    f0 = rf0[pl.ds(corner * _P + o, _L)]
                    f1 = rf1[pl.ds(corner * _P + o, _L)]
                    acc0 = acc0 + f0 * wc
                    acc1 = acc1 + f1 * wc
                f0a[pl.ds(o, _L)] = acc0
                f1a[pl.ds(o, _L)] = acc1

            pltpu.sync_copy(f0a, encT.at[2 * l, pl.ds(base, _P)])
            pltpu.sync_copy(f1a, encT.at[2 * l + 1, pl.ds(base, _P)])


def _encode(xT, tab):
    mesh = plsc.VectorSubcoreMesh(core_axis_name="c", subcore_axis_name="s")
    return pl.kernel(
        _encode_body,
        out_type=jax.ShapeDtypeStruct((2 * _N_LEVELS, _N), jnp.float32),
        mesh=mesh,
        compiler_params=pltpu.CompilerParams(
            use_tc_tiling_on_sc=False, needs_layout_passes=False),
        scratch_types=[
            pltpu.VMEM((_P,), jnp.float32),
            pltpu.VMEM((_P,), jnp.float32),
            pltpu.VMEM((_P,), jnp.float32),
            pltpu.VMEM((_NIDX,), jnp.int32),
            pltpu.VMEM((_NIDX,), jnp.int32),
            pltpu.VMEM((_NIDX,), jnp.float32),
            pltpu.VMEM((_NIDX,), jnp.float32),
            pltpu.VMEM((_NIDX,), jnp.float32),
            pltpu.VMEM((_P,), jnp.float32),
            pltpu.VMEM((_P,), jnp.float32),
            pltpu.SemaphoreType.DMA,
        ],
    )(xT, tab)


_MLP_B = 2048


def _mlp_body(enc_ref, w1t_ref, w2t_ref, out_ref):
    e = enc_ref[...]
    h = jnp.maximum(jnp.dot(w1t_ref[...], e, preferred_element_type=jnp.float32), 0.0)
    o = jnp.dot(w2t_ref[...], h, preferred_element_type=jnp.float32)
    out_ref[...] = jnp.clip(o, 0.0, 1.0)


def _mlp(encT, W1T, W2Tp):
    return pl.pallas_call(
        _mlp_body,
        grid=(_N // _MLP_B,),
        in_specs=[
            pl.BlockSpec((2 * _N_LEVELS, _MLP_B), lambda i: (0, i)),
            pl.BlockSpec((64, 32), lambda i: (0, 0)),
            pl.BlockSpec((8, 64), lambda i: (0, 0)),
        ],
        out_specs=pl.BlockSpec((8, _MLP_B), lambda i: (0, i)),
        out_shape=jax.ShapeDtypeStruct((8, _N), jnp.float32),
    )(encT, W1T, W2Tp)


def kernel(x, table, W1, W2):
    xT = x.T
    tab = table.reshape(_N_LEVELS * _T * 2)
    encT = _encode(xT, tab)
    W1T = W1.T
    W2Tp = jnp.zeros((8, 64), jnp.float32).at[:3].set(W2.T)
    outT = _mlp(encT, W1T, W2Tp)
    return outT[:3].T
